# Initial kernel scaffold; baseline (speedup 1.0000x reference)
import jax
import jax.numpy as jnp
from jax.experimental import pallas as pl


def kernel(x, W_src, W_dst, W_model):
    raise NotImplementedError("write your pallas kernel here")


# speedup vs baseline: 6.7885x; 6.7885x over previous
import functools

import jax
import jax.numpy as jnp
from jax.experimental import pallas as pl

_B, _L, _H = 4, 2048, 4096
_G, _K = 4, 2
_HDG = _H // _G
_DM = _K * _HDG
_T = 512


def _top2(s0, s1, s2, s3):
    neg = jnp.float32(-jnp.inf)

    def top1(a0, a1, a2, a3):
        t01 = a1 > a0
        m01 = jnp.where(t01, a1, a0)
        i01 = jnp.where(t01, 1, 0)
        t23 = a3 > a2
        m23 = jnp.where(t23, a3, a2)
        i23 = jnp.where(t23, 3, 2)
        tf = m23 > m01
        return jnp.where(tf, m23, m01), jnp.where(tf, i23, i01)

    m_a, i_a = top1(s0, s1, s2, s3)
    s0b = jnp.where(i_a == 0, neg, s0)
    s1b = jnp.where(i_a == 1, neg, s1)
    s2b = jnp.where(i_a == 2, neg, s2)
    s3b = jnp.where(i_a == 3, neg, s3)
    m_b, i_b = top1(s0b, s1b, s2b, s3b)
    return m_a, i_a, m_b, i_b


def _mof_kernel(x_ref, wsrc_ref, wdst_ref, wm_ref, out_ref):
    xb = x_ref[...]

    wg = jnp.concatenate([wsrc_ref[...], wdst_ref[...]], axis=0)
    logits = jax.lax.dot_general(
        xb, wg, (((1,), (1,)), ((), ())),
        preferred_element_type=jnp.float32)

    ls = [logits[:, i:i + 1] for i in range(4)]
    ld = [logits[:, 4 + i:5 + i] for i in range(4)]

    ms_a, is_a, ms_b, is_b = _top2(*ls)
    md_a, id_a, md_b, id_b = _top2(*ld)

    gs_a = jax.nn.sigmoid(ms_a)
    gs_b = jax.nn.sigmoid(ms_b)
    gd_a = jax.nn.sigmoid(md_a)
    gd_b = jax.nn.sigmoid(md_b)

    chunks = [xb[:, g * _HDG:(g + 1) * _HDG] for g in range(_G)]
    xa = jnp.zeros_like(chunks[0])
    xbb = jnp.zeros_like(chunks[0])
    for g in range(_G):
        xa = xa + jnp.where(is_a == g, jnp.float32(1), jnp.float32(0)) * chunks[g]
        xbb = xbb + jnp.where(is_b == g, jnp.float32(1), jnp.float32(0)) * chunks[g]
    gathered = jnp.concatenate([gs_a * xa, gs_b * xbb], axis=1)

    y = jax.lax.dot_general(
        gathered, wm_ref[...], (((1,), (1,)), ((), ())),
        preferred_element_type=jnp.float32)

    ya = gd_a * y[:, :_HDG]
    yb = gd_b * y[:, _HDG:]

    for g in range(_G):
        ma = jnp.where(id_a == g, jnp.float32(1), jnp.float32(0))
        mb = jnp.where(id_b == g, jnp.float32(1), jnp.float32(0))
        out_ref[:, g * _HDG:(g + 1) * _HDG] = ma * ya + mb * yb


@functools.partial(jax.jit, static_argnames=())
def kernel(x, W_src, W_dst, W_model):
    b, l, h = x.shape
    n_tok = b * l
    xf = x.reshape(n_tok, h)
    grid = (n_tok // _T,)
    out = pl.pallas_call(
        _mof_kernel,
        grid=grid,
        in_specs=[
            pl.BlockSpec((_T, _H), lambda i: (i, 0)),
            pl.BlockSpec((_G, _H), lambda i: (0, 0)),
            pl.BlockSpec((_G, _H), lambda i: (0, 0)),
            pl.BlockSpec((_DM, _DM), lambda i: (0, 0)),
        ],
        out_specs=pl.BlockSpec((_T, _H), lambda i: (i, 0)),
        out_shape=jax.ShapeDtypeStruct((n_tok, h), jnp.float32),
    )(xf, W_src, W_dst, W_model)
    return out.reshape(b, l, h)
